# Initial kernel scaffold; baseline (speedup 1.0000x reference)
import jax
import jax.numpy as jnp
from jax.experimental import pallas as pl


def kernel(hidden_states, top_k_index, top_k_weights, gate_up_proj, down_proj):
    raise NotImplementedError("write your pallas kernel here")


# speedup vs baseline: 3.8792x; 3.8792x over previous
import functools

import jax
import jax.numpy as jnp
from jax.experimental import pallas as pl
from jax.experimental.pallas import tpu as pltpu


BT = 256


def _routing_metadata(top_k_index, n_exp, n_tok, k):
    S = n_tok * k
    NB = S // BT + n_exp
    NPAD = NB * BT

    e_flat = top_k_index.reshape(-1).astype(jnp.int32)
    onehot = (e_flat[:, None] == jnp.arange(n_exp, dtype=jnp.int32)[None, :])
    csum = jnp.cumsum(onehot.astype(jnp.int32), axis=0)
    counts = csum[-1]
    rank = jnp.take_along_axis(csum, e_flat[:, None], axis=1)[:, 0] - 1
    nb_e = (counts + BT - 1) // BT
    cum_nb = jnp.cumsum(nb_e)
    blk_start = cum_nb - nb_e
    dest = blk_start[e_flat] * BT + rank
    t_flat = jnp.arange(S, dtype=jnp.int32) // k
    tok_pad = jnp.zeros(NPAD, jnp.int32).at[dest].set(t_flat)
    block_expert = jnp.minimum(
        jnp.searchsorted(cum_nb, jnp.arange(NB), side="right"), n_exp - 1
    ).astype(jnp.int32)
    return dest, tok_pad, block_expert, NB, NPAD


def _swiglu_body(I, be_ref, x_ref, w_ref, wt_ref, act_ref):
    x = x_ref[...]
    w = w_ref[0]
    gu = jax.lax.dot_general(
        x, w, (((1,), (1,)), ((), ())), preferred_element_type=jnp.float32
    )
    g = gu[:, :I]
    u = gu[:, I:]
    act = (g * jax.nn.sigmoid(g)) * u
    act_ref[...] = act * wt_ref[...][:, 0:1]


def _down_body(be_ref, act_ref, w_ref, y_ref):
    a = act_ref[...]
    w = w_ref[0]
    y_ref[...] = jax.lax.dot_general(
        a, w, (((1,), (1,)), ((), ())), preferred_element_type=jnp.float32
    )


def kernel(hidden_states, top_k_index, top_k_weights, gate_up_proj, down_proj):
    n_tok, H = hidden_states.shape
    n_exp, twoI, _ = gate_up_proj.shape
    I = twoI // 2
    k = top_k_index.shape[1]

    dest, tok_pad, block_expert, NB, NPAD = _routing_metadata(
        top_k_index, n_exp, n_tok, k
    )

    x_sorted = jnp.take(hidden_states, tok_pad, axis=0)
    wt_pad = (
        jnp.zeros(NPAD, jnp.float32).at[dest].set(top_k_weights.reshape(-1))
    )
    wt_b = jnp.broadcast_to(wt_pad[:, None], (NPAD, 128))

    cparams = pltpu.CompilerParams(
        dimension_semantics=("arbitrary",),
        vmem_limit_bytes=100 * 1024 * 1024,
    )

    act = pl.pallas_call(
        functools.partial(_swiglu_body, I),
        grid_spec=pltpu.PrefetchScalarGridSpec(
            num_scalar_prefetch=1,
            grid=(NB,),
            in_specs=[
                pl.BlockSpec((BT, H), lambda b, be: (b, 0)),
                pl.BlockSpec((1, twoI, H), lambda b, be: (be[b], 0, 0)),
                pl.BlockSpec((BT, 128), lambda b, be: (b, 0)),
            ],
            out_specs=pl.BlockSpec((BT, I), lambda b, be: (b, 0)),
        ),
        out_shape=jax.ShapeDtypeStruct((NPAD, I), jnp.float32),
        compiler_params=cparams,
    )(block_expert, x_sorted, gate_up_proj, wt_b)

    y = pl.pallas_call(
        _down_body,
        grid_spec=pltpu.PrefetchScalarGridSpec(
            num_scalar_prefetch=1,
            grid=(NB,),
            in_specs=[
                pl.BlockSpec((BT, I), lambda b, be: (b, 0)),
                pl.BlockSpec((1, H, I), lambda b, be: (be[b], 0, 0)),
            ],
            out_specs=pl.BlockSpec((BT, H), lambda b, be: (b, 0)),
        ),
        out_shape=jax.ShapeDtypeStruct((NPAD, H), jnp.float32),
        compiler_params=cparams,
    )(block_expert, act, down_proj)

    dest2 = dest.reshape(n_tok, k)
    out = jnp.take(y, dest2[:, 0], axis=0) + jnp.take(y, dest2[:, 1], axis=0)
    return out
